# baseline (speedup 1.0000x reference)
import functools
import math

import jax
import jax.numpy as jnp
from jax.experimental import pallas as pl
from jax.experimental.pallas import tpu as pltpu

_NZ = 100
_NL = 100000
_NU = 1024
_NS = 10
_BK = 1000
_NBLK = _NL // (_NS * _BK)
_LOG_NORM = 0.5 * _NZ * math.log(2.0 * math.pi)


def _min_body(B_ref, L_hbm, o_ref, buf_ref, sem):
    i = pl.program_id(0)
    slot = jax.lax.rem(i, 2)

    def _start(step, into):
        base = step * (_NS * _BK)
        for k in range(_NS):
            pltpu.make_async_copy(
                L_hbm.at[pl.ds(base + k * _BK, _BK), :],
                buf_ref.at[into, k],
                sem.at[into, k],
            ).start()

    @pl.when(i == 0)
    def _():
        _start(0, 0)

    @pl.when(i < _NBLK - 1)
    def _():
        _start(i + 1, 1 - slot)

    B = B_ref[...]
    bmin = None
    for k in range(_NS):
        pltpu.make_async_copy(
            L_hbm.at[pl.ds(0, _BK), :], buf_ref.at[slot, k],
            sem.at[slot, k],
        ).wait()
        Lb = buf_ref[slot, k]
        P = jax.lax.dot_general(
            Lb, B, (((1,), (0,)), ((), ())),
            preferred_element_type=jnp.float32,
        )
        l2 = jnp.sum(Lb * Lb, axis=1, keepdims=True)
        m = jnp.min(l2 + P, axis=0, keepdims=True)
        bmin = m if bmin is None else jnp.minimum(bmin, m)
    o_ref[...] = jnp.where(i == 0, bmin, jnp.minimum(o_ref[...], bmin))


def _fin_body(B_ref, tmin_ref, o_ref):
    B = B_ref[...]
    U2 = 0.25 * jnp.sum(B * B, axis=0, keepdims=True)
    d2 = jnp.maximum(tmin_ref[...] + U2, 0.0)
    div = jnp.log(jnp.sqrt(d2) + 1e-18)
    dens = -0.5 * U2 - _LOG_NORM
    dd = jnp.exp(dens + div)
    dd = dd - jnp.min(dd)
    o_ref[...] = dd / (jnp.max(dd) + 1e-18)


@functools.partial(jax.jit, static_argnames=("interpret",))
def _dd_call(B, L_z, interpret=False):
    tmin = pl.pallas_call(
        _min_body,
        grid=(_NBLK,),
        in_specs=[
            pl.BlockSpec((_NZ, _NU), lambda i: (0, 0)),
            pl.BlockSpec(memory_space=pl.ANY),
        ],
        out_specs=pl.BlockSpec((1, _NU), lambda i: (0, 0)),
        out_shape=jax.ShapeDtypeStruct((1, _NU), jnp.float32),
        scratch_shapes=[
            pltpu.VMEM((2, _NS, _BK, _NZ), jnp.float32),
            pltpu.SemaphoreType.DMA((2, _NS)),
        ],
        compiler_params=pltpu.CompilerParams(
            dimension_semantics=("arbitrary",),
        ),
        interpret=interpret,
    )(B, L_z)
    return pl.pallas_call(
        _fin_body,
        out_shape=jax.ShapeDtypeStruct((1, _NU), jnp.float32),
        interpret=interpret,
    )(B, tmin)


def kernel(pred, U_z, L_z):
    del pred
    out = _dd_call(-2.0 * U_z.T, L_z)
    return out.reshape(-1)


# speedup vs baseline: 1.0567x; 1.0567x over previous
import functools
import math

import jax
import jax.numpy as jnp
from jax.experimental import pallas as pl
from jax.experimental.pallas import tpu as pltpu

_NZ = 100
_NL = 100000
_NU = 1024
_NS = 10
_BK = 1000
_NBLK = _NL // (_NS * _BK)
_LOG_NORM = 0.5 * _NZ * math.log(2.0 * math.pi)


def _min_body(B_ref, *refs):
    (L_refs, o_ref) = (refs[:-1], refs[-1])
    i = pl.program_id(0)
    B16 = B_ref[...].astype(jnp.bfloat16)
    bmin = None
    for Lr in L_refs:
        Lb = Lr[...]
        P = jax.lax.dot_general(
            Lb, B16, (((1,), (0,)), ((), ())),
            preferred_element_type=jnp.float32,
        )
        Lb32 = Lb.astype(jnp.float32)
        l2 = jnp.sum(Lb32 * Lb32, axis=1, keepdims=True)
        m = jnp.min(l2 + P, axis=0, keepdims=True)
        bmin = m if bmin is None else jnp.minimum(bmin, m)
    o_ref[...] = jnp.where(i == 0, bmin, jnp.minimum(o_ref[...], bmin))


def _fin_body(B_ref, tmin_ref, o_ref):
    B = B_ref[...]
    U2 = 0.25 * jnp.sum(B * B, axis=0, keepdims=True)
    d2 = jnp.maximum(tmin_ref[...] + U2, 0.0)
    div = jnp.log(jnp.sqrt(d2) + 1e-18)
    dens = -0.5 * U2 - _LOG_NORM
    dd = jnp.exp(dens + div)
    dd = dd - jnp.min(dd)
    o_ref[...] = dd / (jnp.max(dd) + 1e-18)


@functools.partial(jax.jit, static_argnames=("interpret",))
def _dd_call(B, L_z, interpret=False):
    tmin = pl.pallas_call(
        _min_body,
        grid=(_NBLK,),
        in_specs=[pl.BlockSpec((_NZ, _NU), lambda i: (0, 0))]
        + [pl.BlockSpec((_BK, _NZ), lambda i, k=k: (i * _NS + k, 0))
           for k in range(_NS)],
        out_specs=pl.BlockSpec((1, _NU), lambda i: (0, 0)),
        out_shape=jax.ShapeDtypeStruct((1, _NU), jnp.float32),
        compiler_params=pltpu.CompilerParams(
            dimension_semantics=("arbitrary",),
        ),
        interpret=interpret,
    )(B, *([L_z] * _NS))
    return pl.pallas_call(
        _fin_body,
        out_shape=jax.ShapeDtypeStruct((1, _NU), jnp.float32),
        interpret=interpret,
    )(B, tmin)


def kernel(pred, U_z, L_z):
    del pred
    out = _dd_call(-2.0 * U_z.T, L_z.astype(jnp.bfloat16))
    return out.reshape(-1)


# speedup vs baseline: 1.1457x; 1.0843x over previous
import functools
import math

import jax
import jax.numpy as jnp
from jax.experimental import pallas as pl
from jax.experimental.pallas import tpu as pltpu

_NZ = 100
_NL = 100000
_NU = 1024
_NS = 10
_BK = 1000
_NBLK = _NL // (_NS * _BK)
_LOG_NORM = 0.5 * _NZ * math.log(2.0 * math.pi)


def _min_body(B_ref, *refs):
    (L_refs, o_ref) = (refs[:-1], refs[-1])
    i = pl.program_id(0)
    B = B_ref[...]
    bmin = None
    for Lr in L_refs:
        Lb = Lr[...]
        P = jax.lax.dot_general(
            Lb, B, (((1,), (0,)), ((), ())),
            preferred_element_type=jnp.float32,
        )
        l2 = jnp.sum(Lb * Lb, axis=1, keepdims=True)
        m = jnp.min(l2 + P, axis=0, keepdims=True)
        bmin = m if bmin is None else jnp.minimum(bmin, m)
    o_ref[...] = jnp.where(i == 0, bmin, jnp.minimum(o_ref[...], bmin))


def _fin_body(B_ref, tmin_ref, o_ref):
    B = B_ref[...]
    U2 = 0.25 * jnp.sum(B * B, axis=0, keepdims=True)
    d2 = jnp.maximum(tmin_ref[...] + U2, 0.0)
    div = jnp.log(jnp.sqrt(d2) + 1e-18)
    dens = -0.5 * U2 - _LOG_NORM
    dd = jnp.exp(dens + div)
    dd = dd - jnp.min(dd)
    o_ref[...] = dd / (jnp.max(dd) + 1e-18)


@functools.partial(jax.jit, static_argnames=("interpret",))
def _dd_call(B, L_z, interpret=False):
    tmin = pl.pallas_call(
        _min_body,
        grid=(_NBLK,),
        in_specs=[pl.BlockSpec((_NZ, _NU), lambda i: (0, 0))]
        + [pl.BlockSpec((_BK, _NZ), lambda i, k=k: (i * _NS + k, 0))
           for k in range(_NS)],
        out_specs=pl.BlockSpec((1, _NU), lambda i: (0, 0)),
        out_shape=jax.ShapeDtypeStruct((1, _NU), jnp.float32),
        compiler_params=pltpu.CompilerParams(
            dimension_semantics=("arbitrary",),
        ),
        interpret=interpret,
    )(B, *([L_z] * _NS))
    return pl.pallas_call(
        _fin_body,
        out_shape=jax.ShapeDtypeStruct((1, _NU), jnp.float32),
        interpret=interpret,
    )(B, tmin)


def kernel(pred, U_z, L_z):
    del pred
    out = _dd_call(-2.0 * U_z.T, L_z)
    return out.reshape(-1)
